# Initial kernel scaffold; baseline (speedup 1.0000x reference)
import jax
import jax.numpy as jnp
from jax.experimental import pallas as pl


def kernel(frames):
    raise NotImplementedError("write your pallas kernel here")


# speedup vs baseline: 1.4164x; 1.4164x over previous
import numpy as np
import jax
import jax.numpy as jnp
from jax.experimental import pallas as pl

ALPHA = 4


def _pack_body(in_ref, slow_ref, fast_ref, *, a, b):
    t = pl.program_id(0)
    fast_ref[...] = in_ref[...]
    pos = (b * (t + 1) - 1) // a
    sel = (pos * a) // b == t

    @pl.when(sel)
    def _():
        slow_ref[...] = in_ref[...]


def kernel(frames):
    C, T, H, W = frames.shape
    N = T // ALPHA
    a, b = T - 1, N - 1

    idx = np.linspace(0, T - 1, N).astype(np.int64)
    idx_arith = (np.arange(N) * a) // b
    assert np.array_equal(idx, idx_arith), (idx, idx_arith)

    def slow_index_map(t):
        pos = (b * (t + 1) - 1) // a
        return (0, pos, 0, 0)

    slow, fast = pl.pallas_call(
        lambda i, s, f: _pack_body(i, s, f, a=a, b=b),
        grid=(T,),
        in_specs=[pl.BlockSpec((C, 1, H, W), lambda t: (0, t, 0, 0))],
        out_specs=(
            pl.BlockSpec((C, 1, H, W), slow_index_map),
            pl.BlockSpec((C, 1, H, W), lambda t: (0, t, 0, 0)),
        ),
        out_shape=(
            jax.ShapeDtypeStruct((C, N, H, W), frames.dtype),
            jax.ShapeDtypeStruct((C, T, H, W), frames.dtype),
        ),
    )(frames)
    return (slow, fast)


# speedup vs baseline: 1.6243x; 1.1468x over previous
import numpy as np
import jax
import jax.numpy as jnp
from jax.experimental import pallas as pl

ALPHA = 4


def _pack_body(in_ref, slow_ref, fast_ref, *, a, b):
    s = pl.program_id(0)
    fast_ref[...] = in_ref[...]
    loc = (s * a) // b - ALPHA * s
    slow_ref[...] = in_ref[:, pl.ds(loc, 1)]


def kernel(frames):
    C, T, H, W = frames.shape
    N = T // ALPHA
    a, b = T - 1, N - 1

    idx = np.linspace(0, T - 1, N).astype(np.int64)
    idx_arith = (np.arange(N) * a) // b
    assert np.array_equal(idx, idx_arith), (idx, idx_arith)
    assert np.array_equal(idx // ALPHA, np.arange(N)), idx

    slow, fast = pl.pallas_call(
        lambda i, s, f: _pack_body(i, s, f, a=a, b=b),
        grid=(N,),
        in_specs=[pl.BlockSpec((C, ALPHA, H, W), lambda s: (0, s, 0, 0))],
        out_specs=(
            pl.BlockSpec((C, 1, H, W), lambda s: (0, s, 0, 0)),
            pl.BlockSpec((C, ALPHA, H, W), lambda s: (0, s, 0, 0)),
        ),
        out_shape=(
            jax.ShapeDtypeStruct((C, N, H, W), frames.dtype),
            jax.ShapeDtypeStruct((C, T, H, W), frames.dtype),
        ),
    )(frames)
    return (slow, fast)
